# baseline (speedup 1.0000x reference)
import functools

import jax
import jax.numpy as jnp
from jax import lax
from jax.experimental import pallas as pl
from jax.experimental.pallas import tpu as pltpu
from jax.experimental.pallas import tpu_sc as plsc

H, W = 1080, 1920
NPIX = H * W
NLUT = 33
LUT_C = NLUT * NLUT * NLUT
LUT_WORDS = 3 * LUT_C
NC, NS, L = 2, 16, 16
NW = NC * NS
PIX_PER_TILE = NPIX // NW
P = 2160
NCHUNK = PIX_PER_TILE // P
VPC = P // L

_mesh = plsc.VectorSubcoreMesh(core_axis_name="c", subcore_axis_name="s")


@functools.partial(
    pl.kernel,
    mesh=_mesh,
    compiler_params=pltpu.CompilerParams(needs_layout_passes=False),
    out_type=jax.ShapeDtypeStruct((3 * NPIX,), jnp.float32),
    scratch_types=[
        pltpu.VMEM((LUT_WORDS,), jnp.int32),
        pltpu.VMEM((P,), jnp.float32),
        pltpu.VMEM((P,), jnp.float32),
        pltpu.VMEM((P,), jnp.float32),
        pltpu.VMEM((P,), jnp.float32),
        pltpu.VMEM((P,), jnp.float32),
        pltpu.VMEM((P,), jnp.float32),
    ],
)
def _interp(lut_hbm, img_hbm, out_hbm, lut_v, r_v, g_v, b_v, o0_v, o1_v, o2_v):
    wid = lax.axis_index("s") * NC + lax.axis_index("c")
    pltpu.sync_copy(lut_hbm, lut_v)
    tile_base = wid * PIX_PER_TILE

    def chunk_body(ci, carry):
        start = tile_base + ci * P
        pltpu.sync_copy(img_hbm.at[pl.ds(start, P)], r_v)
        pltpu.sync_copy(img_hbm.at[pl.ds(NPIX + start, P)], g_v)
        pltpu.sync_copy(img_hbm.at[pl.ds(2 * NPIX + start, P)], b_v)

        def vec_body(j, c2):
            o = j * L
            r = r_v[pl.ds(o, L)]
            g = g_v[pl.ds(o, L)]
            b = b_v[pl.ds(o, L)]
            fx = jnp.minimum(jnp.maximum(r * 32.0, 0.0), 32.0)
            fy = jnp.minimum(jnp.maximum(g * 32.0, 0.0), 32.0)
            fz = jnp.minimum(jnp.maximum(b * 32.0, 0.0), 32.0)
            ix0 = fx.astype(jnp.int32)
            iy0 = fy.astype(jnp.int32)
            iz0 = fz.astype(jnp.int32)
            wx = fx - ix0.astype(jnp.float32)
            wy = fy - iy0.astype(jnp.float32)
            wz = fz - iz0.astype(jnp.float32)
            ux = 1.0 - wx
            dy = (jnp.minimum(iy0 + 1, NLUT - 1) - iy0) * NLUT
            dz = (jnp.minimum(iz0 + 1, NLUT - 1) - iz0) * (NLUT * NLUT)
            dyz = dy + dz
            base = iz0 * (NLUT * NLUT) + iy0 * NLUT + ix0

            def pairval(idx):
                w = plsc.load_gather(lut_v, [idx])
                lo = plsc.bitcast(w << 16, jnp.float32)
                hi = plsc.bitcast(w & jnp.int32(-65536), jnp.float32)
                return lo * ux + hi * wx

            for coff, o_ref in ((0, o0_v), (LUT_C, o1_v), (2 * LUT_C, o2_v)):
                c00 = base + coff
                v00 = pairval(c00)
                v01 = pairval(c00 + dy)
                v10 = pairval(c00 + dz)
                v11 = pairval(c00 + dyz)
                vy0 = v00 + wy * (v01 - v00)
                vy1 = v10 + wy * (v11 - v10)
                o_ref[pl.ds(o, L)] = vy0 + wz * (vy1 - vy0)
            return c2

        lax.fori_loop(0, VPC, vec_body, 0)
        pltpu.sync_copy(o0_v, out_hbm.at[pl.ds(start, P)])
        pltpu.sync_copy(o1_v, out_hbm.at[pl.ds(NPIX + start, P)])
        pltpu.sync_copy(o2_v, out_hbm.at[pl.ds(2 * NPIX + start, P)])
        return carry

    lax.fori_loop(0, NCHUNK, chunk_body, 0)


def _pack_lut(lut):
    lo = lut
    hi = jnp.concatenate([lut[..., 1:], lut[..., NLUT - 1:]], axis=-1)
    lo16 = lax.bitcast_convert_type(lo.astype(jnp.bfloat16), jnp.uint16)
    hi16 = lax.bitcast_convert_type(hi.astype(jnp.bfloat16), jnp.uint16)
    packed = (hi16.astype(jnp.uint32) << 16) | lo16.astype(jnp.uint32)
    return lax.bitcast_convert_type(packed, jnp.int32).reshape(LUT_WORDS)


def kernel(lut, img):
    img_flat = img.reshape(3 * NPIX)
    out = _interp(_pack_lut(lut), img_flat)
    return (lut[None], out.reshape(1, 3, H, W))


# speedup vs baseline: 1.2891x; 1.2891x over previous
import functools

import jax
import jax.numpy as jnp
from jax import lax
from jax.experimental import pallas as pl
from jax.experimental.pallas import tpu as pltpu
from jax.experimental.pallas import tpu_sc as plsc

H, W = 1080, 1920
NPIX = H * W
NLUT = 33
LUT_C = NLUT * NLUT * NLUT
LUT_WORDS = 3 * LUT_C
NC, NS, L = 2, 16, 16
NW = NC * NS
PIX_PER_TILE = NPIX // NW
P = 2160
NCHUNK = PIX_PER_TILE // P
VPC = P // L

_mesh = plsc.VectorSubcoreMesh(core_axis_name="c", subcore_axis_name="s")


@functools.partial(
    pl.kernel,
    mesh=_mesh,
    compiler_params=pltpu.CompilerParams(needs_layout_passes=False),
    out_type=jax.ShapeDtypeStruct((3 * NPIX,), jnp.float32),
    scratch_types=[
        pltpu.VMEM((LUT_WORDS,), jnp.int32),
        pltpu.VMEM((P,), jnp.float32),
        pltpu.VMEM((P,), jnp.float32),
        pltpu.VMEM((P,), jnp.float32),
        pltpu.VMEM((P,), jnp.float32),
        pltpu.VMEM((P,), jnp.float32),
        pltpu.VMEM((P,), jnp.float32),
    ],
)
def _interp(lut_hbm, img_hbm, out_hbm, lut_v, r_v, g_v, b_v, o0_v, o1_v, o2_v):
    wid = lax.axis_index("s") * NC + lax.axis_index("c")
    pltpu.sync_copy(lut_hbm, lut_v)
    tile_base = wid * PIX_PER_TILE

    def chunk_body(ci, carry):
        start = tile_base + ci * P
        pltpu.sync_copy(img_hbm.at[pl.ds(start, P)], r_v)
        pltpu.sync_copy(img_hbm.at[pl.ds(NPIX + start, P)], g_v)
        pltpu.sync_copy(img_hbm.at[pl.ds(2 * NPIX + start, P)], b_v)

        @plsc.parallel_loop(0, VPC, 1, unroll=4)
        def vec_body(j):
            o = j * L
            r = r_v[pl.ds(o, L)]
            g = g_v[pl.ds(o, L)]
            b = b_v[pl.ds(o, L)]
            fx = jnp.minimum(jnp.maximum(r * 32.0, 0.0), 32.0)
            fy = jnp.minimum(jnp.maximum(g * 32.0, 0.0), 32.0)
            fz = jnp.minimum(jnp.maximum(b * 32.0, 0.0), 32.0)
            ix0 = fx.astype(jnp.int32)
            iy0 = fy.astype(jnp.int32)
            iz0 = fz.astype(jnp.int32)
            wx = fx - ix0.astype(jnp.float32)
            wy = fy - iy0.astype(jnp.float32)
            wz = fz - iz0.astype(jnp.float32)
            ux = 1.0 - wx
            dy = (jnp.minimum(iy0 + 1, NLUT - 1) - iy0) * NLUT
            dz = (jnp.minimum(iz0 + 1, NLUT - 1) - iz0) * (NLUT * NLUT)
            dyz = dy + dz
            base = iz0 * (NLUT * NLUT) + iy0 * NLUT + ix0

            def pairval(idx):
                w = plsc.load_gather(lut_v, [idx])
                lo = plsc.bitcast(w << 16, jnp.float32)
                hi = plsc.bitcast(w & jnp.int32(-65536), jnp.float32)
                return lo * ux + hi * wx

            for coff, o_ref in ((0, o0_v), (LUT_C, o1_v), (2 * LUT_C, o2_v)):
                c00 = base + coff
                v00 = pairval(c00)
                v01 = pairval(c00 + dy)
                v10 = pairval(c00 + dz)
                v11 = pairval(c00 + dyz)
                vy0 = v00 + wy * (v01 - v00)
                vy1 = v10 + wy * (v11 - v10)
                o_ref[pl.ds(o, L)] = vy0 + wz * (vy1 - vy0)

        pltpu.sync_copy(o0_v, out_hbm.at[pl.ds(start, P)])
        pltpu.sync_copy(o1_v, out_hbm.at[pl.ds(NPIX + start, P)])
        pltpu.sync_copy(o2_v, out_hbm.at[pl.ds(2 * NPIX + start, P)])
        return carry

    lax.fori_loop(0, NCHUNK, chunk_body, 0)


def _pack_lut(lut):
    lo = lut
    hi = jnp.concatenate([lut[..., 1:], lut[..., NLUT - 1:]], axis=-1)
    lo16 = lax.bitcast_convert_type(lo.astype(jnp.bfloat16), jnp.uint16)
    hi16 = lax.bitcast_convert_type(hi.astype(jnp.bfloat16), jnp.uint16)
    packed = (hi16.astype(jnp.uint32) << 16) | lo16.astype(jnp.uint32)
    return lax.bitcast_convert_type(packed, jnp.int32).reshape(LUT_WORDS)


def kernel(lut, img):
    img_flat = img.reshape(3 * NPIX)
    out = _interp(_pack_lut(lut), img_flat)
    return (lut[None], out.reshape(1, 3, H, W))


# speedup vs baseline: 1.6484x; 1.2788x over previous
import functools

import jax
import jax.numpy as jnp
from jax import lax
from jax.experimental import pallas as pl
from jax.experimental.pallas import tpu as pltpu
from jax.experimental.pallas import tpu_sc as plsc

H, W = 1080, 1920
NPIX = H * W
NLUT = 33
LUT_C = NLUT * NLUT * NLUT
LUT_WORDS = 3 * LUT_C
NC, NS, L = 2, 16, 16
NW = NC * NS
PIX_PER_TILE = NPIX // NW
P = 1200
NCHUNK = PIX_PER_TILE // P
NPAIR = NCHUNK // 2
VPC = P // L

_mesh = plsc.VectorSubcoreMesh(core_axis_name="c", subcore_axis_name="s")


@functools.partial(
    pl.kernel,
    mesh=_mesh,
    compiler_params=pltpu.CompilerParams(needs_layout_passes=False),
    out_type=jax.ShapeDtypeStruct((3 * NPIX,), jnp.float32),
    scratch_types=[
        pltpu.VMEM((LUT_WORDS,), jnp.int32),
    ]
    + [pltpu.VMEM((P,), jnp.float32) for _ in range(12)]
    + [pltpu.SemaphoreType.DMA for _ in range(4)],
)
def _interp(lut_hbm, img_hbm, out_hbm, lut_v,
            ra, ga, ba, rb, gb, bb, oa0, oa1, oa2, ob0, ob1, ob2,
            in_sa, in_sb, out_sa, out_sb):
    wid = lax.axis_index("s") * NC + lax.axis_index("c")
    tile_base = wid * PIX_PER_TILE

    def issue_in(ci, rd, gd, bd, sem):
        start = tile_base + ci * P
        pltpu.async_copy(img_hbm.at[pl.ds(start, P)], rd, sem)
        pltpu.async_copy(img_hbm.at[pl.ds(NPIX + start, P)], gd, sem)
        pltpu.async_copy(img_hbm.at[pl.ds(2 * NPIX + start, P)], bd, sem)

    def wait_in(rd, gd, bd, sem):
        for d in (rd, gd, bd):
            pltpu.make_async_copy(img_hbm.at[pl.ds(0, P)], d, sem).wait()

    def issue_out(ci, o0, o1, o2, sem):
        start = tile_base + ci * P
        pltpu.async_copy(o0, out_hbm.at[pl.ds(start, P)], sem)
        pltpu.async_copy(o1, out_hbm.at[pl.ds(NPIX + start, P)], sem)
        pltpu.async_copy(o2, out_hbm.at[pl.ds(2 * NPIX + start, P)], sem)

    def wait_out(o0, o1, o2, sem):
        for d in (o0, o1, o2):
            pltpu.make_async_copy(d, out_hbm.at[pl.ds(0, P)], sem).wait()

    def compute(r_v, g_v, b_v, o0_v, o1_v, o2_v):
        @plsc.parallel_loop(0, VPC, 1, unroll=4)
        def vec_body(j):
            o = j * L
            r = r_v[pl.ds(o, L)]
            g = g_v[pl.ds(o, L)]
            b = b_v[pl.ds(o, L)]
            fx = jnp.minimum(jnp.maximum(r * 32.0, 0.0), 32.0)
            fy = jnp.minimum(jnp.maximum(g * 32.0, 0.0), 32.0)
            fz = jnp.minimum(jnp.maximum(b * 32.0, 0.0), 32.0)
            ix0 = fx.astype(jnp.int32)
            iy0 = fy.astype(jnp.int32)
            iz0 = fz.astype(jnp.int32)
            wx = fx - ix0.astype(jnp.float32)
            wy = fy - iy0.astype(jnp.float32)
            wz = fz - iz0.astype(jnp.float32)
            ux = 1.0 - wx
            dy = (jnp.minimum(iy0 + 1, NLUT - 1) - iy0) * NLUT
            dz = (jnp.minimum(iz0 + 1, NLUT - 1) - iz0) * (NLUT * NLUT)
            dyz = dy + dz
            base = iz0 * (NLUT * NLUT) + iy0 * NLUT + ix0

            def pairval(idx):
                w = plsc.load_gather(lut_v, [idx])
                lo = plsc.bitcast(w << 16, jnp.float32)
                hi = plsc.bitcast(w & jnp.int32(-65536), jnp.float32)
                return lo * ux + hi * wx

            for coff, o_ref in ((0, o0_v), (LUT_C, o1_v), (2 * LUT_C, o2_v)):
                c00 = base + coff
                v00 = pairval(c00)
                v01 = pairval(c00 + dy)
                v10 = pairval(c00 + dz)
                v11 = pairval(c00 + dyz)
                vy0 = v00 + wy * (v01 - v00)
                vy1 = v10 + wy * (v11 - v10)
                o_ref[pl.ds(o, L)] = vy0 + wz * (vy1 - vy0)

    issue_in(0, ra, ga, ba, in_sa)
    pltpu.sync_copy(lut_hbm, lut_v)

    def pair_body(k, carry):
        c0 = 2 * k
        issue_in(c0 + 1, rb, gb, bb, in_sb)
        wait_in(ra, ga, ba, in_sa)
        compute(ra, ga, ba, oa0, oa1, oa2)
        issue_out(c0, oa0, oa1, oa2, out_sa)
        issue_in(jnp.minimum(c0 + 2, NCHUNK - 1), ra, ga, ba, in_sa)
        wait_in(rb, gb, bb, in_sb)
        compute(rb, gb, bb, ob0, ob1, ob2)
        issue_out(c0 + 1, ob0, ob1, ob2, out_sb)
        wait_out(oa0, oa1, oa2, out_sa)
        wait_out(ob0, ob1, ob2, out_sb)
        return carry

    lax.fori_loop(0, NPAIR, pair_body, 0)
    wait_in(ra, ga, ba, in_sa)


def _pack_lut(lut):
    lo = lut
    hi = jnp.concatenate([lut[..., 1:], lut[..., NLUT - 1:]], axis=-1)
    lo16 = lax.bitcast_convert_type(lo.astype(jnp.bfloat16), jnp.uint16)
    hi16 = lax.bitcast_convert_type(hi.astype(jnp.bfloat16), jnp.uint16)
    packed = (hi16.astype(jnp.uint32) << 16) | lo16.astype(jnp.uint32)
    return lax.bitcast_convert_type(packed, jnp.int32).reshape(LUT_WORDS)


def kernel(lut, img):
    img_flat = img.reshape(3 * NPIX)
    out = _interp(_pack_lut(lut), img_flat)
    return (lut[None], out.reshape(1, 3, H, W))
